# baseline (speedup 1.0000x reference)
import numpy as np

import jax
import jax.numpy as jnp
from jax.experimental import pallas as pl

N = 10
E = 45
A = 4
D = 32
ITERS = 8

TB = 512

_EF, _ET = (list(int(v) for v in ix) for ix in np.triu_indices(N, k=1))
_S = [0] + list(np.cumsum([N - 1 - n for n in range(N)])[:N - 1])
_S = [int(v) for v in _S]

_DEFAULT = jax.lax.Precision.DEFAULT
_HIGHEST = jax.lax.Precision.HIGHEST


def _dcg_body(x_ref, we_ref, be_ref, wn_ref, bn_ref, gf_ref, gt_ref,
              qmax_ref, amax_ref):
    x = x_ref[...]
    we = we_ref[...]

    outs = []
    for e in range(E):
        xin = jnp.concatenate(
            [x[_EF[e] * D:(_EF[e] + 1) * D, :],
             x[_ET[e] * D:(_ET[e] + 1) * D, :]], axis=0)
        outs.append(jax.lax.dot_general(
            we[e * A * A:(e + 1) * A * A, :], xin,
            (((1,), (0,)), ((), ())), precision=_DEFAULT,
            preferred_element_type=jnp.float32))
    ev3 = jnp.stack(outs, axis=1)
    ev = [[ev3[ai * A + aj] + be_ref[ai * A + aj]
           for aj in range(A)] for ai in range(A)]
    evE = [[ev[ai][aj] * (1.0 / E) for aj in range(A)] for ai in range(A)]

    wn = wn_ref[...]
    nouts = [jax.lax.dot_general(
        wn[n * A:(n + 1) * A, :], x[n * D:(n + 1) * D, :],
        (((1,), (0,)), ((), ())), precision=_DEFAULT,
        preferred_element_type=jnp.float32) for n in range(N)]
    nv3 = jnp.stack(nouts, axis=1)
    nv = [nv3[a] + bn_ref[a] for a in range(A)]
    nvN = [v * (1.0 / N) for v in nv]

    gf = gf_ref[...]
    gt = gt_ref[...]

    def mm(m, v):
        return jax.lax.dot_general(m, v, (((1,), (0,)), ((), ())),
                                   precision=_HIGHEST,
                                   preferred_element_type=jnp.float32)

    def gather_from(p):
        return jnp.concatenate(
            [jnp.broadcast_to(p[n:n + 1, :], (N - 1 - n, TB))
             for n in range(N - 1)], axis=0)

    def gather_to(p):
        return jnp.concatenate([p[n + 1:N, :] for n in range(N - 1)], axis=0)

    def argmax_mask(planes):
        m = jnp.maximum(jnp.maximum(planes[0], planes[1]),
                        jnp.maximum(planes[2], planes[3]))
        e0 = planes[0] >= m
        e1 = (planes[1] >= m) & ~e0
        e2 = (planes[2] >= m) & ~(e0 | e1)
        e3 = ~(e0 | e1 | e2)
        return [e0.astype(jnp.float32), e1.astype(jnp.float32),
                e2.astype(jnp.float32), e3.astype(jnp.float32)]

    def eval_a(amask):
        nterm = amask[0] * nv[0]
        for a in range(1, A):
            nterm = nterm + amask[a] * nv[a]
        nterm = jnp.sum(nterm, axis=0, keepdims=True)
        afm = [mm(gf, amask[a]) for a in range(A)]
        atm = [mm(gt, amask[a]) for a in range(A)]
        eterm = None
        for ai in range(A):
            inner = atm[0] * ev[ai][0]
            for aj in range(1, A):
                inner = inner + atm[aj] * ev[ai][aj]
            contrib = afm[ai] * inner
            eterm = contrib if eterm is None else eterm + contrib
        eterm = jnp.sum(eterm, axis=0, keepdims=True)
        return nterm * (1.0 / N) + eterm * (1.0 / E)

    def mask_to_idx(amask):
        return amask[1] + 2.0 * amask[2] + 3.0 * amask[3]

    amask = argmax_mask(nv)
    q_max = eval_a(amask)
    a_idx = mask_to_idx(amask)

    mf = [jnp.zeros((E, TB), jnp.float32) for _ in range(A)]
    mb = [jnp.zeros((E, TB), jnp.float32) for _ in range(A)]
    q = list(nvN)

    for _ in range(ITERS):
        qf = [gather_from(q[a]) for a in range(A)]
        qt = [gather_to(q[a]) for a in range(A)]
        tf = [qf[a] - mb[a] for a in range(A)]
        tb = [qt[a] - mf[a] for a in range(A)]
        nmf, nmb = [], []
        for y in range(A):
            acc = tf[0] + evE[0][y]
            for xx in range(1, A):
                acc = jnp.maximum(acc, tf[xx] + evE[xx][y])
            nmf.append(acc)
            acc = tb[0] + evE[y][0]
            for xx in range(1, A):
                acc = jnp.maximum(acc, tb[xx] + evE[y][xx])
            nmb.append(acc)
        msf = ((nmf[0] + nmf[2]) + (nmf[1] + nmf[3])) * (1.0 / A)
        msb = ((nmb[0] + nmb[2]) + (nmb[1] + nmb[3])) * (1.0 / A)
        mf = [v - msf for v in nmf]
        mb = [v - msb for v in nmb]
        q = []
        zrow = jnp.zeros((1, TB), jnp.float32)
        for a in range(A):
            qa = nvN[a]
            for k in range(N - 1):
                slab = jnp.concatenate(
                    [jnp.zeros((k + 1, TB), jnp.float32),
                     mf[a][_S[k]:_S[k] + N - 1 - k, :]], axis=0)
                qa = qa + slab
            for k in range(N - 1):
                rows = [mb[a][_S[n] + k:_S[n] + k + 1, :] if k <= N - 2 - n
                        else zrow for n in range(N)]
                qa = qa + jnp.concatenate(rows, axis=0)
            q.append(qa)
        amask = argmax_mask(q)
        q_val = eval_a(amask)
        upd = q_val > q_max
        q_max = jnp.where(upd, q_val, q_max)
        a_idx = jnp.where(upd, mask_to_idx(amask), a_idx)

    qmax_ref[...] = q_max
    amax_ref[...] = a_idx.astype(jnp.int32)


@jax.jit
def kernel(obs, W_node, b_node, W_edge, b_edge, edges_from, edges_to):
    bsz = obs.shape[0]
    f32 = jnp.float32

    ohf = jax.nn.one_hot(edges_from, N, dtype=f32)
    oht = jax.nn.one_hot(edges_to, N, dtype=f32)

    obs_t = obs.reshape(bsz, N * D).T
    w_et = jnp.swapaxes(W_edge, 1, 2).reshape(E * A * A, 2 * D)
    b_ep = b_edge.T.reshape(A * A, E, 1)
    wn = jnp.swapaxes(W_node, 1, 2).reshape(N * A, D)
    b_n = b_node.T.reshape(A, N, 1)

    grid = (bsz // TB,)
    qmax, amax = pl.pallas_call(
        _dcg_body,
        grid=grid,
        in_specs=[
            pl.BlockSpec((N * D, TB), lambda i: (0, i)),
            pl.BlockSpec((E * A * A, 2 * D), lambda i: (0, 0)),
            pl.BlockSpec((A * A, E, 1), lambda i: (0, 0, 0)),
            pl.BlockSpec((N * A, D), lambda i: (0, 0)),
            pl.BlockSpec((A, N, 1), lambda i: (0, 0, 0)),
            pl.BlockSpec((E, N), lambda i: (0, 0)),
            pl.BlockSpec((E, N), lambda i: (0, 0)),
        ],
        out_specs=[
            pl.BlockSpec((1, TB), lambda i: (0, i)),
            pl.BlockSpec((N, TB), lambda i: (0, i)),
        ],
        out_shape=[
            jax.ShapeDtypeStruct((1, bsz), f32),
            jax.ShapeDtypeStruct((N, bsz), jnp.int32),
        ],
    )(obs_t, w_et, b_ep, wn, b_n, ohf, oht)
    return qmax.reshape(bsz), amax.T


# speedup vs baseline: 1.0124x; 1.0124x over previous
import numpy as np

import jax
import jax.numpy as jnp
from jax.experimental import pallas as pl

N = 10
E = 45
A = 4
D = 32
ITERS = 8

TB = 128

_EF, _ET = (list(int(v) for v in ix) for ix in np.triu_indices(N, k=1))
_S = [0] + list(np.cumsum([N - 1 - n for n in range(N)])[:N - 1])
_S = [int(v) for v in _S]

_DEFAULT = jax.lax.Precision.DEFAULT
_HIGHEST = jax.lax.Precision.HIGHEST


def _dcg_body(x_ref, we_ref, be_ref, wn_ref, bn_ref, gf_ref, gt_ref,
              qmax_ref, amax_ref):
    x = x_ref[...]
    we = we_ref[...]

    outs = []
    for e in range(E):
        xin = jnp.concatenate(
            [x[_EF[e] * D:(_EF[e] + 1) * D, :],
             x[_ET[e] * D:(_ET[e] + 1) * D, :]], axis=0)
        outs.append(jax.lax.dot_general(
            we[e * A * A:(e + 1) * A * A, :], xin,
            (((1,), (0,)), ((), ())), precision=_DEFAULT,
            preferred_element_type=jnp.float32))
    ev3 = jnp.stack(outs, axis=1)
    ev = [[ev3[ai * A + aj] + be_ref[ai * A + aj]
           for aj in range(A)] for ai in range(A)]
    evE = [[ev[ai][aj] * (1.0 / E) for aj in range(A)] for ai in range(A)]

    wn = wn_ref[...]
    nouts = [jax.lax.dot_general(
        wn[n * A:(n + 1) * A, :], x[n * D:(n + 1) * D, :],
        (((1,), (0,)), ((), ())), precision=_DEFAULT,
        preferred_element_type=jnp.float32) for n in range(N)]
    nv3 = jnp.stack(nouts, axis=1)
    nv = [nv3[a] + bn_ref[a] for a in range(A)]
    nvN = [v * (1.0 / N) for v in nv]

    gf = gf_ref[...]
    gt = gt_ref[...]

    def mm(m, v):
        return jax.lax.dot_general(m, v, (((1,), (0,)), ((), ())),
                                   precision=_HIGHEST,
                                   preferred_element_type=jnp.float32)

    def gather_from(p):
        return jnp.concatenate(
            [jnp.broadcast_to(p[n:n + 1, :], (N - 1 - n, TB))
             for n in range(N - 1)], axis=0)

    def gather_to(p):
        return jnp.concatenate([p[n + 1:N, :] for n in range(N - 1)], axis=0)

    def argmax_mask(planes):
        m = jnp.maximum(jnp.maximum(planes[0], planes[1]),
                        jnp.maximum(planes[2], planes[3]))
        e0 = planes[0] >= m
        e1 = (planes[1] >= m) & ~e0
        e2 = (planes[2] >= m) & ~(e0 | e1)
        e3 = ~(e0 | e1 | e2)
        return [e0.astype(jnp.float32), e1.astype(jnp.float32),
                e2.astype(jnp.float32), e3.astype(jnp.float32)]

    def eval_a(amask):
        nterm = amask[0] * nv[0]
        for a in range(1, A):
            nterm = nterm + amask[a] * nv[a]
        nterm = jnp.sum(nterm, axis=0, keepdims=True)
        afm = [mm(gf, amask[a]) for a in range(A)]
        atm = [mm(gt, amask[a]) for a in range(A)]
        eterm = None
        for ai in range(A):
            inner = atm[0] * ev[ai][0]
            for aj in range(1, A):
                inner = inner + atm[aj] * ev[ai][aj]
            contrib = afm[ai] * inner
            eterm = contrib if eterm is None else eterm + contrib
        eterm = jnp.sum(eterm, axis=0, keepdims=True)
        return nterm * (1.0 / N) + eterm * (1.0 / E)

    def mask_to_idx(amask):
        return amask[1] + 2.0 * amask[2] + 3.0 * amask[3]

    amask = argmax_mask(nv)
    q_max = eval_a(amask)
    a_idx = mask_to_idx(amask)

    mf = [jnp.zeros((E, TB), jnp.float32) for _ in range(A)]
    mb = [jnp.zeros((E, TB), jnp.float32) for _ in range(A)]
    q = list(nvN)

    for _ in range(ITERS):
        qf = [gather_from(q[a]) for a in range(A)]
        qt = [gather_to(q[a]) for a in range(A)]
        tf = [qf[a] - mb[a] for a in range(A)]
        tb = [qt[a] - mf[a] for a in range(A)]
        nmf, nmb = [], []
        for y in range(A):
            acc = tf[0] + evE[0][y]
            for xx in range(1, A):
                acc = jnp.maximum(acc, tf[xx] + evE[xx][y])
            nmf.append(acc)
            acc = tb[0] + evE[y][0]
            for xx in range(1, A):
                acc = jnp.maximum(acc, tb[xx] + evE[y][xx])
            nmb.append(acc)
        msf = ((nmf[0] + nmf[2]) + (nmf[1] + nmf[3])) * (1.0 / A)
        msb = ((nmb[0] + nmb[2]) + (nmb[1] + nmb[3])) * (1.0 / A)
        mf = [v - msf for v in nmf]
        mb = [v - msb for v in nmb]
        q = []
        zrow = jnp.zeros((1, TB), jnp.float32)
        for a in range(A):
            qa = nvN[a]
            for k in range(N - 1):
                slab = jnp.concatenate(
                    [jnp.zeros((k + 1, TB), jnp.float32),
                     mf[a][_S[k]:_S[k] + N - 1 - k, :]], axis=0)
                qa = qa + slab
            for k in range(N - 1):
                rows = [mb[a][_S[n] + k:_S[n] + k + 1, :] if k <= N - 2 - n
                        else zrow for n in range(N)]
                qa = qa + jnp.concatenate(rows, axis=0)
            q.append(qa)
        amask = argmax_mask(q)
        q_val = eval_a(amask)
        upd = q_val > q_max
        q_max = jnp.where(upd, q_val, q_max)
        a_idx = jnp.where(upd, mask_to_idx(amask), a_idx)

    qmax_ref[...] = q_max
    amax_ref[...] = a_idx.astype(jnp.int32)


@jax.jit
def kernel(obs, W_node, b_node, W_edge, b_edge, edges_from, edges_to):
    bsz = obs.shape[0]
    f32 = jnp.float32

    ohf = jax.nn.one_hot(edges_from, N, dtype=f32)
    oht = jax.nn.one_hot(edges_to, N, dtype=f32)

    obs_t = obs.reshape(bsz, N * D).T
    w_et = jnp.swapaxes(W_edge, 1, 2).reshape(E * A * A, 2 * D)
    b_ep = b_edge.T.reshape(A * A, E, 1)
    wn = jnp.swapaxes(W_node, 1, 2).reshape(N * A, D)
    b_n = b_node.T.reshape(A, N, 1)

    grid = (bsz // TB,)
    qmax, amax = pl.pallas_call(
        _dcg_body,
        grid=grid,
        in_specs=[
            pl.BlockSpec((N * D, TB), lambda i: (0, i)),
            pl.BlockSpec((E * A * A, 2 * D), lambda i: (0, 0)),
            pl.BlockSpec((A * A, E, 1), lambda i: (0, 0, 0)),
            pl.BlockSpec((N * A, D), lambda i: (0, 0)),
            pl.BlockSpec((A, N, 1), lambda i: (0, 0, 0)),
            pl.BlockSpec((E, N), lambda i: (0, 0)),
            pl.BlockSpec((E, N), lambda i: (0, 0)),
        ],
        out_specs=[
            pl.BlockSpec((1, TB), lambda i: (0, i)),
            pl.BlockSpec((N, TB), lambda i: (0, i)),
        ],
        out_shape=[
            jax.ShapeDtypeStruct((1, bsz), f32),
            jax.ShapeDtypeStruct((N, bsz), jnp.int32),
        ],
    )(obs_t, w_et, b_ep, wn, b_n, ohf, oht)
    return qmax.reshape(bsz), amax.T


# speedup vs baseline: 1.2840x; 1.2683x over previous
import numpy as np

import jax
import jax.numpy as jnp
from jax.experimental import pallas as pl

N = 10
E = 45
A = 4
D = 32
ITERS = 8

TB = 256

_EF, _ET = (list(int(v) for v in ix) for ix in np.triu_indices(N, k=1))
_S = [0] + list(np.cumsum([N - 1 - n for n in range(N)])[:N - 1])
_S = [int(v) for v in _S]

_DEFAULT = jax.lax.Precision.DEFAULT


def _dcg_body(x_ref, we_ref, be_ref, wn_ref, bn_ref, qmax_ref, amax_ref):
    x = x_ref[...]
    we = we_ref[...]

    outs = []
    for e in range(E):
        xin = jnp.concatenate(
            [x[_EF[e] * D:(_EF[e] + 1) * D, :],
             x[_ET[e] * D:(_ET[e] + 1) * D, :]], axis=0)
        outs.append(jax.lax.dot_general(
            we[e * A * A:(e + 1) * A * A, :], xin,
            (((1,), (0,)), ((), ())), precision=_DEFAULT,
            preferred_element_type=jnp.float32))
    ev3 = jnp.stack(outs, axis=1)
    ev = [[ev3[ai * A + aj] + be_ref[ai * A + aj]
           for aj in range(A)] for ai in range(A)]
    evE = [[ev[ai][aj] * (1.0 / E) for aj in range(A)] for ai in range(A)]

    wn = wn_ref[...]
    nouts = [jax.lax.dot_general(
        wn[n * A:(n + 1) * A, :], x[n * D:(n + 1) * D, :],
        (((1,), (0,)), ((), ())), precision=_DEFAULT,
        preferred_element_type=jnp.float32) for n in range(N)]
    nv3 = jnp.stack(nouts, axis=1)
    nv = [nv3[a] + bn_ref[a] for a in range(A)]
    nvN = [v * (1.0 / N) for v in nv]

    def gather_from(p):
        return jnp.concatenate(
            [jnp.broadcast_to(p[n:n + 1, :], (N - 1 - n, TB))
             for n in range(N - 1)], axis=0)

    def gather_to(p):
        return jnp.concatenate([p[n + 1:N, :] for n in range(N - 1)], axis=0)

    def argmax_mask(planes):
        m = jnp.maximum(jnp.maximum(planes[0], planes[1]),
                        jnp.maximum(planes[2], planes[3]))
        e0 = planes[0] >= m
        e1 = (planes[1] >= m) & ~e0
        e2 = (planes[2] >= m) & ~(e0 | e1)
        return [e0, e1, e2]

    def sel4(masks, vals):
        return jnp.where(masks[0], vals[0],
                         jnp.where(masks[1], vals[1],
                                   jnp.where(masks[2], vals[2], vals[3])))

    def eval_a(masks, idxp):
        nterm = jnp.sum(sel4(masks, nv), axis=0, keepdims=True)
        afc = gather_from(idxp)
        atc = gather_to(idxp)
        afm = [afc == 0.0, afc == 1.0, afc == 2.0]
        atm = [atc == 0.0, atc == 1.0, atc == 2.0]
        inner = [sel4(atm, ev[ai]) for ai in range(A)]
        eterm = jnp.sum(sel4(afm, inner), axis=0, keepdims=True)
        return nterm * (1.0 / N) + eterm * (1.0 / E)

    def mask_to_idx(masks):
        one = jnp.full((N, TB), 1.0, jnp.float32)
        two = jnp.full((N, TB), 2.0, jnp.float32)
        three = jnp.full((N, TB), 3.0, jnp.float32)
        zero = jnp.zeros((N, TB), jnp.float32)
        return jnp.where(masks[1], one,
                         jnp.where(masks[2], two,
                                   jnp.where(masks[0], zero, three)))

    amask = argmax_mask(nv)
    a_idx = mask_to_idx(amask)
    q_max = eval_a(amask, a_idx)

    mf = [jnp.zeros((E, TB), jnp.float32) for _ in range(A)]
    mb = [jnp.zeros((E, TB), jnp.float32) for _ in range(A)]
    q = list(nvN)

    for _ in range(ITERS):
        qf = [gather_from(q[a]) for a in range(A)]
        qt = [gather_to(q[a]) for a in range(A)]
        tf = [qf[a] - mb[a] for a in range(A)]
        tb = [qt[a] - mf[a] for a in range(A)]
        nmf, nmb = [], []
        for y in range(A):
            acc = tf[0] + evE[0][y]
            for xx in range(1, A):
                acc = jnp.maximum(acc, tf[xx] + evE[xx][y])
            nmf.append(acc)
            acc = tb[0] + evE[y][0]
            for xx in range(1, A):
                acc = jnp.maximum(acc, tb[xx] + evE[y][xx])
            nmb.append(acc)
        msf = ((nmf[0] + nmf[2]) + (nmf[1] + nmf[3])) * (1.0 / A)
        msb = ((nmb[0] + nmb[2]) + (nmb[1] + nmb[3])) * (1.0 / A)
        mf = [v - msf for v in nmf]
        mb = [v - msb for v in nmb]
        q = []
        for a in range(A):
            qa = nvN[a]
            for k in range(N - 1):
                slab = jnp.concatenate(
                    [jnp.zeros((k + 1, TB), jnp.float32),
                     mf[a][_S[k]:_S[k] + N - 1 - k, :]], axis=0)
                qa = qa + slab
            rows = []
            for n in range(N):
                acc = qa[n:n + 1, :]
                for k in range(N - 1 - n):
                    acc = acc + mb[a][_S[n] + k:_S[n] + k + 1, :]
                rows.append(acc)
            q.append(jnp.concatenate(rows, axis=0))
        amask = argmax_mask(q)
        idxp = mask_to_idx(amask)
        q_val = eval_a(amask, idxp)
        upd = q_val > q_max
        q_max = jnp.where(upd, q_val, q_max)
        a_idx = jnp.where(upd, idxp, a_idx)

    qmax_ref[...] = q_max
    amax_ref[...] = a_idx.astype(jnp.int32)


@jax.jit
def kernel(obs, W_node, b_node, W_edge, b_edge, edges_from, edges_to):
    bsz = obs.shape[0]
    f32 = jnp.float32

    obs_t = obs.reshape(bsz, N * D).T
    w_et = jnp.swapaxes(W_edge, 1, 2).reshape(E * A * A, 2 * D)
    b_ep = b_edge.T.reshape(A * A, E, 1)
    wn = jnp.swapaxes(W_node, 1, 2).reshape(N * A, D)
    b_n = b_node.T.reshape(A, N, 1)

    grid = (bsz // TB,)
    qmax, amax = pl.pallas_call(
        _dcg_body,
        grid=grid,
        in_specs=[
            pl.BlockSpec((N * D, TB), lambda i: (0, i)),
            pl.BlockSpec((E * A * A, 2 * D), lambda i: (0, 0)),
            pl.BlockSpec((A * A, E, 1), lambda i: (0, 0, 0)),
            pl.BlockSpec((N * A, D), lambda i: (0, 0)),
            pl.BlockSpec((A, N, 1), lambda i: (0, 0, 0)),
        ],
        out_specs=[
            pl.BlockSpec((1, TB), lambda i: (0, i)),
            pl.BlockSpec((N, TB), lambda i: (0, i)),
        ],
        out_shape=[
            jax.ShapeDtypeStruct((1, bsz), f32),
            jax.ShapeDtypeStruct((N, bsz), jnp.int32),
        ],
    )(obs_t, w_et, b_ep, wn, b_n)
    return qmax.reshape(bsz), amax.T
